# baseline (speedup 1.0000x reference)
import functools

import jax
import jax.numpy as jnp
from jax import lax
from jax.experimental import pallas as pl
from jax.experimental.pallas import tpu as pltpu
from jax.experimental.pallas import tpu_sc as plsc

HIDDEN = 2048
NUM_EXPERTS = 8
TOP_K = 2
F_TEXT = 1024
SHARED_F = 2048
NORM_MIN = 1e-12
T = 2048
BM = 128
NT = T // BM
NSLOT = T * TOP_K
BMG = 256
NTILES = 24
NROWS = NTILES * BMG

HP = HIDDEN // 2
NC = 2
NS = 16
NW = NC * NS
SLOTS_PER_W = NSLOT // NW
CH = 32
NCHUNK = SLOTS_PER_W // CH


def _shift_down(c, sh):
    return jnp.concatenate([jnp.zeros((sh,) + c.shape[1:], c.dtype), c[:-sh]], axis=0)


def _router_body(x_ref, wt_ref, bias_ref, logits_ref, pos_ref, rw_ref, te_ref):
    x = x_ref[...]
    wt = wt_ref[...]
    logits = jnp.dot(x, wt, preferred_element_type=jnp.float32)
    logits_ref[...] = logits
    m = jnp.max(logits, axis=1, keepdims=True)
    ex = jnp.exp(logits - m)
    probs = ex / jnp.sum(ex, axis=1, keepdims=True)
    corrected = probs + bias_ref[...]
    idx = jax.lax.broadcasted_iota(jnp.int32, corrected.shape, 1)
    big = jnp.int32(NUM_EXPERTS)

    def top1(c):
        m1 = jnp.max(c, axis=1, keepdims=True)
        return jnp.min(jnp.where(c == m1, idx, big), axis=1, keepdims=True)

    a1 = top1(corrected)
    a2 = top1(jnp.where(idx == a1, -jnp.inf, corrected))
    p1 = jnp.sum(jnp.where(idx == a1, probs, 0.0), axis=1, keepdims=True)
    p2 = jnp.sum(jnp.where(idx == a2, probs, 0.0), axis=1, keepdims=True)
    denom = jnp.maximum(p1 + p2, NORM_MIN)
    rw_ref[...] = jnp.concatenate([p1 / denom, p2 / denom], axis=0)

    s_cat = jnp.concatenate([a1, a2], axis=0)
    oh = (s_cat == jax.lax.broadcasted_iota(jnp.int32, (NSLOT, NUM_EXPERTS), 1)
          ).astype(jnp.int32)
    incl = oh
    sh = 1
    while sh < NSLOT:
        incl = incl + _shift_down(incl, sh)
        sh *= 2
    rank = jnp.sum(jnp.where(oh == 1, incl, 0), axis=1, keepdims=True) - 1
    counts = incl[NSLOT - 1:NSLOT, :]
    ptiles = (counts + (BMG - 1)) // BMG

    def lane_cumsum(v):
        for s in (1, 2, 4):
            v = v + jnp.concatenate(
                [jnp.zeros((1, s), v.dtype), v[:, :-s]], axis=1)
        return v

    tiles_incl = lane_cumsum(ptiles)
    rows_excl = (tiles_incl - ptiles) * BMG
    pos_ref[...] = jnp.sum(jnp.where(oh == 1, rows_excl, 0),
                           axis=1, keepdims=True) + rank
    mi = jax.lax.broadcasted_iota(jnp.int32, (NTILES, NUM_EXPERTS), 0)
    te = jnp.sum((mi >= tiles_incl).astype(jnp.int32), axis=1, keepdims=True)
    te_ref[...] = jnp.minimum(te, NUM_EXPERTS - 1)


_SC_SCRATCH = [
    pltpu.VMEM((CH,), jnp.int32),
    pltpu.VMEM((CH, HIDDEN), jnp.float32),
    pltpu.SemaphoreType.DMA,
]


@functools.partial(
    pl.kernel,
    mesh=plsc.VectorSubcoreMesh(core_axis_name="c", subcore_axis_name="s"),
    out_type=jax.ShapeDtypeStruct((NROWS, HIDDEN), jnp.float32),
    scratch_types=_SC_SCRATCH,
)
def _sc_scatter(x_hbm, pos_hbm, xs_hbm, idx_v, rows_v, sem):
    wid = lax.axis_index("s") * NC + lax.axis_index("c")
    base = wid * SLOTS_PER_W
    for r in range(NCHUNK):
        i0 = base + r * CH
        pltpu.sync_copy(pos_hbm.at[pl.ds(i0, CH)], idx_v)
        tok0 = lax.rem(i0, T)
        pltpu.sync_copy(x_hbm.at[pl.ds(tok0, CH)], rows_v)
        pltpu.async_copy(rows_v, xs_hbm.at[idx_v], sem).wait()


@functools.partial(
    pl.kernel,
    mesh=plsc.VectorSubcoreMesh(core_axis_name="c", subcore_axis_name="s"),
    out_type=jax.ShapeDtypeStruct((NSLOT, HIDDEN), jnp.float32),
    scratch_types=_SC_SCRATCH,
)
def _sc_gather(exp_hbm, pos_hbm, eg_hbm, idx_v, rows_v, sem):
    wid = lax.axis_index("s") * NC + lax.axis_index("c")
    base = wid * SLOTS_PER_W
    for r in range(NCHUNK):
        i0 = base + r * CH
        pltpu.sync_copy(pos_hbm.at[pl.ds(i0, CH)], idx_v)
        pltpu.async_copy(exp_hbm.at[idx_v], rows_v, sem).wait()
        pltpu.sync_copy(rows_v, eg_hbm.at[pl.ds(i0, CH)])


def _gmm_body(te_ref, xs_ref, gu_ref, dn_ref, out_ref):
    x = xs_ref[...]
    gu = jnp.dot(x, gu_ref[0], preferred_element_type=jnp.float32)
    g = gu[:, :F_TEXT]
    u = gu[:, F_TEXT:]
    h = (g * jax.nn.sigmoid(g)) * u
    out_ref[...] = jnp.dot(h, dn_ref[0], preferred_element_type=jnp.float32)


def _dot_t(a, b):
    return lax.dot_general(a, b, (((1,), (1,)), ((), ())),
                           preferred_element_type=jnp.float32)


def _shared_body(x_ref, gw_ref, uw_ref, dw_ref, eg0_ref, eg1_ref,
                 w0_ref, w1_ref, out_ref):
    xt = x_ref[...]
    g = _dot_t(xt, gw_ref[...])
    u = _dot_t(xt, uw_ref[...])
    h = (g * jax.nn.sigmoid(g)) * u
    out_ref[...] = (_dot_t(h, dw_ref[...]) + w0_ref[...] * eg0_ref[...]
                    + w1_ref[...] * eg1_ref[...])


@jax.jit
def _run(x, router_weight, e_score_bias, gate_up_proj, down_proj,
         shared_gate_w, shared_up_w, shared_down_w):
    xf = x.reshape(T, HIDDEN)
    logits, pos, rw_cat, te = pl.pallas_call(
        _router_body,
        grid=(1,),
        in_specs=[
            pl.BlockSpec((T, HIDDEN), lambda i: (0, 0)),
            pl.BlockSpec((HIDDEN, NUM_EXPERTS), lambda i: (0, 0)),
            pl.BlockSpec((1, NUM_EXPERTS), lambda i: (0, 0)),
        ],
        out_specs=[
            pl.BlockSpec((T, NUM_EXPERTS), lambda i: (0, 0)),
            pl.BlockSpec((NSLOT, 1), lambda i: (0, 0)),
            pl.BlockSpec((NSLOT, 1), lambda i: (0, 0)),
            pl.BlockSpec((NTILES, 1), lambda i: (0, 0)),
        ],
        out_shape=[
            jax.ShapeDtypeStruct((T, NUM_EXPERTS), jnp.float32),
            jax.ShapeDtypeStruct((NSLOT, 1), jnp.int32),
            jax.ShapeDtypeStruct((NSLOT, 1), jnp.float32),
            jax.ShapeDtypeStruct((NTILES, 1), jnp.int32),
        ],
    )(xf, router_weight.T, e_score_bias)

    pos_f = pos.reshape(NSLOT)
    xs = _sc_scatter(xf, pos_f)

    expanded = pl.pallas_call(
        _gmm_body,
        grid_spec=pltpu.PrefetchScalarGridSpec(
            num_scalar_prefetch=1,
            grid=(NTILES,),
            in_specs=[
                pl.BlockSpec((BMG, HIDDEN), lambda m, te_r: (m, 0)),
                pl.BlockSpec((1, HIDDEN, 2 * F_TEXT), lambda m, te_r: (te_r[m], 0, 0)),
                pl.BlockSpec((1, F_TEXT, HIDDEN), lambda m, te_r: (te_r[m], 0, 0)),
            ],
            out_specs=pl.BlockSpec((BMG, HIDDEN), lambda m, te_r: (m, 0)),
        ),
        out_shape=jax.ShapeDtypeStruct((NROWS, HIDDEN), jnp.float32),
        compiler_params=pltpu.CompilerParams(
            dimension_semantics=("arbitrary",),
        ),
    )(te.reshape(NTILES), xs, gate_up_proj, down_proj)

    eg = _sc_gather(expanded, pos_f)

    final = pl.pallas_call(
        _shared_body,
        grid=(NT,),
        in_specs=[
            pl.BlockSpec((BM, HIDDEN), lambda t: (t, 0)),
            pl.BlockSpec((SHARED_F, HIDDEN), lambda t: (0, 0)),
            pl.BlockSpec((SHARED_F, HIDDEN), lambda t: (0, 0)),
            pl.BlockSpec((HIDDEN, SHARED_F), lambda t: (0, 0)),
            pl.BlockSpec((BM, HIDDEN), lambda t: (t, 0)),
            pl.BlockSpec((BM, HIDDEN), lambda t: (t + NT, 0)),
            pl.BlockSpec((BM, 1), lambda t: (t, 0)),
            pl.BlockSpec((BM, 1), lambda t: (t + NT, 0)),
        ],
        out_specs=pl.BlockSpec((BM, HIDDEN), lambda t: (t, 0)),
        out_shape=jax.ShapeDtypeStruct((T, HIDDEN), jnp.float32),
    )(xf, shared_gate_w, shared_up_w, shared_down_w, eg, eg, rw_cat, rw_cat)

    return final.reshape(1, T, HIDDEN), logits


def kernel(hidden_states, router_weight, e_score_bias, gate_up_proj, down_proj,
           shared_gate_w, shared_up_w, shared_down_w):
    return _run(hidden_states, router_weight, e_score_bias, gate_up_proj,
                down_proj, shared_gate_w, shared_up_w, shared_down_w)


# speedup vs baseline: 1.2229x; 1.2229x over previous
import functools

import jax
import jax.numpy as jnp
from jax import lax
from jax.experimental import pallas as pl
from jax.experimental.pallas import tpu as pltpu
from jax.experimental.pallas import tpu_sc as plsc

HIDDEN = 2048
NUM_EXPERTS = 8
TOP_K = 2
F_TEXT = 1024
SHARED_F = 2048
NORM_MIN = 1e-12
T = 2048
BM = 256
NT = T // BM
NSLOT = T * TOP_K
BMG = 128
NTILES = 40
NROWS = NTILES * BMG

HP = HIDDEN // 2
NC = 2
NS = 16
NW = NC * NS
SLOTS_PER_W = NSLOT // NW
CH = 32
NCHUNK = SLOTS_PER_W // CH


def _shift_down(c, sh):
    return jnp.concatenate([jnp.zeros((sh,) + c.shape[1:], c.dtype), c[:-sh]], axis=0)


def _router_body(x_ref, wt_ref, bias_ref, logits_ref, pos_ref, rw_ref, te_ref):
    x = x_ref[...]
    wt = wt_ref[...]
    logits = jnp.dot(x, wt, preferred_element_type=jnp.float32)
    logits_ref[...] = logits
    m = jnp.max(logits, axis=1, keepdims=True)
    ex = jnp.exp(logits - m)
    probs = ex / jnp.sum(ex, axis=1, keepdims=True)
    corrected = probs + bias_ref[...]
    idx = jax.lax.broadcasted_iota(jnp.int32, corrected.shape, 1)
    big = jnp.int32(NUM_EXPERTS)

    def top1(c):
        m1 = jnp.max(c, axis=1, keepdims=True)
        return jnp.min(jnp.where(c == m1, idx, big), axis=1, keepdims=True)

    a1 = top1(corrected)
    a2 = top1(jnp.where(idx == a1, -jnp.inf, corrected))
    p1 = jnp.sum(jnp.where(idx == a1, probs, 0.0), axis=1, keepdims=True)
    p2 = jnp.sum(jnp.where(idx == a2, probs, 0.0), axis=1, keepdims=True)
    denom = jnp.maximum(p1 + p2, NORM_MIN)
    rw_ref[...] = jnp.concatenate([p1 / denom, p2 / denom], axis=0)

    s_cat = jnp.concatenate([a1, a2], axis=0)
    oh = (s_cat == jax.lax.broadcasted_iota(jnp.int32, (NSLOT, NUM_EXPERTS), 1)
          ).astype(jnp.int32)
    incl = oh
    sh = 1
    while sh < NSLOT:
        incl = incl + _shift_down(incl, sh)
        sh *= 2
    rank = jnp.sum(jnp.where(oh == 1, incl, 0), axis=1, keepdims=True) - 1
    counts = incl[NSLOT - 1:NSLOT, :]
    ptiles = (counts + (BMG - 1)) // BMG

    def lane_cumsum(v):
        for s in (1, 2, 4):
            v = v + jnp.concatenate(
                [jnp.zeros((1, s), v.dtype), v[:, :-s]], axis=1)
        return v

    tiles_incl = lane_cumsum(ptiles)
    rows_excl = (tiles_incl - ptiles) * BMG
    pos_ref[...] = jnp.sum(jnp.where(oh == 1, rows_excl, 0),
                           axis=1, keepdims=True) + rank
    mi = jax.lax.broadcasted_iota(jnp.int32, (NTILES, NUM_EXPERTS), 0)
    te = jnp.sum((mi >= tiles_incl).astype(jnp.int32), axis=1, keepdims=True)
    te_ref[...] = jnp.minimum(te, NUM_EXPERTS - 1)


_SC_SCRATCH = [
    pltpu.VMEM((CH,), jnp.int32),
    pltpu.VMEM((CH, HIDDEN), jnp.float32),
    pltpu.SemaphoreType.DMA,
]


@functools.partial(
    pl.kernel,
    mesh=plsc.VectorSubcoreMesh(core_axis_name="c", subcore_axis_name="s"),
    out_type=jax.ShapeDtypeStruct((NROWS, HIDDEN), jnp.float32),
    scratch_types=_SC_SCRATCH,
)
def _sc_scatter(x_hbm, pos_hbm, xs_hbm, idx_v, rows_v, sem):
    wid = lax.axis_index("s") * NC + lax.axis_index("c")
    base = wid * SLOTS_PER_W
    for r in range(NCHUNK):
        i0 = base + r * CH
        pltpu.sync_copy(pos_hbm.at[pl.ds(i0, CH)], idx_v)
        tok0 = lax.rem(i0, T)
        pltpu.sync_copy(x_hbm.at[pl.ds(tok0, CH)], rows_v)
        pltpu.async_copy(rows_v, xs_hbm.at[idx_v], sem).wait()


@functools.partial(
    pl.kernel,
    mesh=plsc.VectorSubcoreMesh(core_axis_name="c", subcore_axis_name="s"),
    out_type=jax.ShapeDtypeStruct((NSLOT, HIDDEN), jnp.float32),
    scratch_types=_SC_SCRATCH,
)
def _sc_gather(exp_hbm, pos_hbm, eg_hbm, idx_v, rows_v, sem):
    wid = lax.axis_index("s") * NC + lax.axis_index("c")
    base = wid * SLOTS_PER_W
    for r in range(NCHUNK):
        i0 = base + r * CH
        pltpu.sync_copy(pos_hbm.at[pl.ds(i0, CH)], idx_v)
        pltpu.async_copy(exp_hbm.at[idx_v], rows_v, sem).wait()
        pltpu.sync_copy(rows_v, eg_hbm.at[pl.ds(i0, CH)])


def _gmm_body(te_ref, xs_ref, gu_ref, dn_ref, out_ref):
    x = xs_ref[...]
    gu = jnp.dot(x, gu_ref[0], preferred_element_type=jnp.float32)
    g = gu[:, :F_TEXT]
    u = gu[:, F_TEXT:]
    h = (g * jax.nn.sigmoid(g)) * u
    out_ref[...] = jnp.dot(h, dn_ref[0], preferred_element_type=jnp.float32)


def _dot_t(a, b):
    return lax.dot_general(a, b, (((1,), (1,)), ((), ())),
                           preferred_element_type=jnp.float32)


def _shared_body(x_ref, gw_ref, uw_ref, dw_ref, out_ref):
    xt = x_ref[...]
    g = _dot_t(xt, gw_ref[...])
    u = _dot_t(xt, uw_ref[...])
    h = (g * jax.nn.sigmoid(g)) * u
    out_ref[...] = _dot_t(h, dw_ref[...])


def _combine_body(sh_ref, eg0_ref, eg1_ref, w0_ref, w1_ref, out_ref):
    out_ref[...] = (sh_ref[...] + w0_ref[...] * eg0_ref[...]
                    + w1_ref[...] * eg1_ref[...])


@jax.jit
def _run(x, router_weight, e_score_bias, gate_up_proj, down_proj,
         shared_gate_w, shared_up_w, shared_down_w):
    xf = x.reshape(T, HIDDEN)
    logits, pos, rw_cat, te = pl.pallas_call(
        _router_body,
        grid=(1,),
        in_specs=[
            pl.BlockSpec((T, HIDDEN), lambda i: (0, 0)),
            pl.BlockSpec((HIDDEN, NUM_EXPERTS), lambda i: (0, 0)),
            pl.BlockSpec((1, NUM_EXPERTS), lambda i: (0, 0)),
        ],
        out_specs=[
            pl.BlockSpec((T, NUM_EXPERTS), lambda i: (0, 0)),
            pl.BlockSpec((NSLOT, 1), lambda i: (0, 0)),
            pl.BlockSpec((NSLOT, 1), lambda i: (0, 0)),
            pl.BlockSpec((NTILES, 1), lambda i: (0, 0)),
        ],
        out_shape=[
            jax.ShapeDtypeStruct((T, NUM_EXPERTS), jnp.float32),
            jax.ShapeDtypeStruct((NSLOT, 1), jnp.int32),
            jax.ShapeDtypeStruct((NSLOT, 1), jnp.float32),
            jax.ShapeDtypeStruct((NTILES, 1), jnp.int32),
        ],
    )(xf, router_weight.T, e_score_bias)

    pos_f = pos.reshape(NSLOT)
    xs = _sc_scatter(xf, pos_f)

    shared_out = pl.pallas_call(
        _shared_body,
        grid=(NT,),
        in_specs=[
            pl.BlockSpec((BM, HIDDEN), lambda t: (t, 0)),
            pl.BlockSpec((SHARED_F, HIDDEN), lambda t: (0, 0)),
            pl.BlockSpec((SHARED_F, HIDDEN), lambda t: (0, 0)),
            pl.BlockSpec((HIDDEN, SHARED_F), lambda t: (0, 0)),
        ],
        out_specs=pl.BlockSpec((BM, HIDDEN), lambda t: (t, 0)),
        out_shape=jax.ShapeDtypeStruct((T, HIDDEN), jnp.float32),
    )(xf, shared_gate_w, shared_up_w, shared_down_w)

    expanded = pl.pallas_call(
        _gmm_body,
        grid_spec=pltpu.PrefetchScalarGridSpec(
            num_scalar_prefetch=1,
            grid=(NTILES,),
            in_specs=[
                pl.BlockSpec((BMG, HIDDEN), lambda m, te_r: (m, 0)),
                pl.BlockSpec((1, HIDDEN, 2 * F_TEXT), lambda m, te_r: (te_r[m], 0, 0)),
                pl.BlockSpec((1, F_TEXT, HIDDEN), lambda m, te_r: (te_r[m], 0, 0)),
            ],
            out_specs=pl.BlockSpec((BMG, HIDDEN), lambda m, te_r: (m, 0)),
        ),
        out_shape=jax.ShapeDtypeStruct((NROWS, HIDDEN), jnp.float32),
        compiler_params=pltpu.CompilerParams(
            dimension_semantics=("arbitrary",),
        ),
    )(te.reshape(NTILES), xs, gate_up_proj, down_proj)

    eg = _sc_gather(expanded, pos_f)

    final = pl.pallas_call(
        _combine_body,
        grid=(NT,),
        in_specs=[
            pl.BlockSpec((BM, HIDDEN), lambda t: (t, 0)),
            pl.BlockSpec((BM, HIDDEN), lambda t: (t, 0)),
            pl.BlockSpec((BM, HIDDEN), lambda t: (t + NT, 0)),
            pl.BlockSpec((BM, 1), lambda t: (t, 0)),
            pl.BlockSpec((BM, 1), lambda t: (t + NT, 0)),
        ],
        out_specs=pl.BlockSpec((BM, HIDDEN), lambda t: (t, 0)),
        out_shape=jax.ShapeDtypeStruct((T, HIDDEN), jnp.float32),
    )(shared_out, eg, eg, rw_cat, rw_cat)

    return final.reshape(1, T, HIDDEN), logits


def kernel(hidden_states, router_weight, e_score_bias, gate_up_proj, down_proj,
           shared_gate_w, shared_up_w, shared_down_w):
    return _run(hidden_states, router_weight, e_score_bias, gate_up_proj,
                down_proj, shared_gate_w, shared_up_w, shared_down_w)
